# baseline (speedup 1.0000x reference)
import functools
import math

import jax
import jax.numpy as jnp
from jax import lax
from jax.experimental import pallas as pl
from jax.experimental.pallas import tpu as pltpu
from jax.experimental.pallas import tpu_sc as plsc

D_MODEL = 64
SCALE = math.sqrt(D_MODEL)

NC = 2
NS = 16
NW = NC * NS

IB = 128
L = 16


@jax.jit
def _embed(xp, table):
    nj_hi, nt, nj_lo, ib = xp.shape
    nj = nj_hi * nj_lo
    assert nt == NW and ib == IB
    mesh = plsc.VectorSubcoreMesh(core_axis_name="c", subcore_axis_name="s")

    @functools.partial(
        pl.kernel,
        mesh=mesh,
        out_type=jax.ShapeDtypeStruct(
            (nj, D_MODEL // 8, NW, 8, IB), jnp.float32
        ),
        scratch_types=[
            pltpu.VMEM((nj_hi, nj_lo, IB), jnp.int32),
            pltpu.VMEM((IB, D_MODEL), jnp.float32),
            pltpu.VMEM((IB, D_MODEL), jnp.float32),
            pltpu.VMEM((D_MODEL // 8, 8, IB), jnp.float32),
            pltpu.VMEM((D_MODEL // 8, 8, IB), jnp.float32),
            pltpu.SemaphoreType.DMA,
            pltpu.SemaphoreType.DMA,
            pltpu.SemaphoreType.DMA,
            pltpu.SemaphoreType.DMA,
        ],
        compiler_params=pltpu.CompilerParams(
            use_tc_tiling_on_sc=False, needs_layout_passes=False
        ),
    )
    def k(xp_hbm, table_hbm, out_hbm,
          xv, rows0, rows1, blk0, blk1, gsem0, gsem1, wsem0, wsem1):
        t = lax.axis_index("s") * NC + lax.axis_index("c")
        rows = (rows0, rows1)
        blks = (blk0, blk1)
        gsems = (gsem0, gsem1)
        wsems = (wsem0, wsem1)

        pltpu.sync_copy(xp_hbm.at[:, t], xv)

        iota = lax.iota(jnp.int32, L)

        def fire(j, b):
            pltpu.async_copy(
                table_hbm.at[xv.at[j // nj_lo, j % nj_lo]],
                rows[b],
                gsems[b],
            )

        def drain_gather(b):
            pltpu.make_async_copy(
                table_hbm.at[pl.ds(0, IB)], rows[b], gsems[b]
            ).wait()

        def drain_wb(b):
            pltpu.make_async_copy(
                out_hbm.at[0, :, 0], blks[b], wsems[b]
            ).wait()

        def transpose_scale(b):
            for g in range(IB // L):
                row_idx = iota + g * L

                @plsc.parallel_loop(0, D_MODEL, unroll=8)
                def _(kf):
                    col_idx = jnp.full((L,), kf, jnp.int32)
                    vals = plsc.load_gather(rows[b], [row_idx, col_idx])
                    blks[b][kf // 8, kf % 8, pl.ds(g * L, L)] = vals * SCALE

        fire(0, 0)

        def outer(g2, carry):
            for b in range(2):
                j = g2 * 2 + b

                @pl.when(j + 1 < nj)
                def _():
                    fire(j + 1, 1 - b)

                drain_gather(b)

                @pl.when(j >= 2)
                def _():
                    drain_wb(b)

                transpose_scale(b)
                pltpu.async_copy(blks[b], out_hbm.at[j, :, t], wsems[b])
            return carry

        lax.fori_loop(0, nj // 2, outer, 0)
        drain_wb(0)
        drain_wb(1)

    return k(xp, table)


def kernel(x, table):
    n_batch, n_seq = x.shape
    ni = n_batch // IB
    xp = (
        x.T.reshape(n_seq // 8, 8, ni, IB)
        .transpose(0, 2, 1, 3)
        .astype(jnp.int32)
    )
    out5 = _embed(xp, table)
    return out5.transpose(2, 4, 0, 1, 3).reshape(n_batch, n_seq, D_MODEL)


# speedup vs baseline: 1.0027x; 1.0027x over previous
import functools
import math

import jax
import jax.numpy as jnp
from jax import lax
from jax.experimental import pallas as pl
from jax.experimental.pallas import tpu as pltpu
from jax.experimental.pallas import tpu_sc as plsc

D_MODEL = 64
SCALE = math.sqrt(D_MODEL)

NC = 2
NS = 16
NW = NC * NS

IB = 128
L = 16
RING = 4


@jax.jit
def _embed(xp, table):
    nj_hi, nt, nj_lo, ib = xp.shape
    nj = nj_hi * nj_lo
    assert nt == NW and ib == IB and nj % RING == 0
    mesh = plsc.VectorSubcoreMesh(core_axis_name="c", subcore_axis_name="s")

    @functools.partial(
        pl.kernel,
        mesh=mesh,
        out_type=jax.ShapeDtypeStruct(
            (nj, D_MODEL // 8, NW, 8 * IB), jnp.float32
        ),
        scratch_types=[
            pltpu.VMEM((nj_hi, nj_lo, IB), jnp.int32),
            pltpu.VMEM((IB, D_MODEL), jnp.float32),
            pltpu.VMEM((IB, D_MODEL), jnp.float32),
            pltpu.VMEM((IB, D_MODEL), jnp.float32),
            pltpu.VMEM((IB, D_MODEL), jnp.float32),
            pltpu.VMEM((D_MODEL // 8, 8 * IB), jnp.float32),
            pltpu.VMEM((D_MODEL // 8, 8 * IB), jnp.float32),
            pltpu.SemaphoreType.DMA,
            pltpu.SemaphoreType.DMA,
            pltpu.SemaphoreType.DMA,
            pltpu.SemaphoreType.DMA,
            pltpu.SemaphoreType.DMA,
            pltpu.SemaphoreType.DMA,
        ],
        compiler_params=pltpu.CompilerParams(
            use_tc_tiling_on_sc=False, needs_layout_passes=False
        ),
    )
    def k(xp_hbm, table_hbm, out_hbm,
          xv, r0, r1, r2, r3, b0, b1, g0, g1, g2, g3, w0, w1):
        t = lax.axis_index("s") * NC + lax.axis_index("c")
        rows = (r0, r1, r2, r3)
        blks = (b0, b1)
        gsems = (g0, g1, g2, g3)
        wsems = (w0, w1)

        pltpu.sync_copy(xp_hbm.at[:, t], xv)

        iota = lax.iota(jnp.int32, L)

        def fire(j, s):
            pltpu.async_copy(
                table_hbm.at[xv.at[j // nj_lo, j % nj_lo]],
                rows[s],
                gsems[s],
            )

        def drain_gather(s):
            pltpu.make_async_copy(
                table_hbm.at[pl.ds(0, IB)], rows[s], gsems[s]
            ).wait()

        def drain_wb(p):
            pltpu.make_async_copy(
                out_hbm.at[0, :, 0], blks[p], wsems[p]
            ).wait()

        def transpose_scale(s, p):
            for g in range(IB // L):
                row_idx = iota + g * L

                @plsc.parallel_loop(0, D_MODEL, unroll=8)
                def _(kf):
                    col_idx = jnp.full((L,), kf, jnp.int32)
                    vals = plsc.load_gather(rows[s], [row_idx, col_idx])
                    off = (kf & 7) * IB + g * L
                    blks[p][kf >> 3, pl.ds(off, L)] = vals * SCALE

        for s in range(RING - 1):
            fire(s, s)

        def outer(g4, carry):
            for r in range(RING):
                j = g4 * RING + r

                @pl.when(j + RING - 1 < nj)
                def _():
                    fire(j + RING - 1, (r + RING - 1) % RING)

                drain_gather(r)
                p = r % 2

                @pl.when(j >= 2)
                def _():
                    drain_wb(p)

                transpose_scale(r, p)
                pltpu.async_copy(blks[p], out_hbm.at[j, :, t], wsems[p])
            return carry

        lax.fori_loop(0, nj // RING, outer, 0)
        drain_wb(0)
        drain_wb(1)

    return k(xp, table)


def kernel(x, table):
    n_batch, n_seq = x.shape
    ni = n_batch // IB
    xp = (
        x.T.reshape(n_seq // 8, 8, ni, IB)
        .transpose(0, 2, 1, 3)
        .astype(jnp.int32)
    )
    out5 = _embed(xp, table)
    return (
        out5.reshape(n_seq, D_MODEL // 8, ni, 8, IB)
        .transpose(2, 4, 0, 1, 3)
        .reshape(n_batch, n_seq, D_MODEL)
    )


# speedup vs baseline: 1.6881x; 1.6835x over previous
import functools
import math

import jax
import jax.numpy as jnp
from jax import lax
from jax.experimental import pallas as pl
from jax.experimental.pallas import tpu as pltpu
from jax.experimental.pallas import tpu_sc as plsc

D_MODEL = 64
SCALE = math.sqrt(D_MODEL)

NC = 2
NS = 16
NW = NC * NS

IB = 128
L = 16
RING = 4


@jax.jit
def _embed(xp, table):
    nj_hi, nt, nj_lo, ib = xp.shape
    nj = nj_hi * nj_lo
    assert nt == NW and ib == IB and nj % RING == 0
    mesh = plsc.VectorSubcoreMesh(core_axis_name="c", subcore_axis_name="s")

    @functools.partial(
        pl.kernel,
        mesh=mesh,
        out_type=jax.ShapeDtypeStruct(
            (nj, D_MODEL // 8, NW, 8, IB), jnp.float32
        ),
        scratch_types=[
            pltpu.VMEM((nj_hi, nj_lo, IB), jnp.int32),
            pltpu.VMEM((IB, D_MODEL), jnp.float32),
            pltpu.VMEM((IB, D_MODEL), jnp.float32),
            pltpu.VMEM((IB, D_MODEL), jnp.float32),
            pltpu.VMEM((IB, D_MODEL), jnp.float32),
            pltpu.VMEM((D_MODEL // 8, 8, IB + 1), jnp.float32),
            pltpu.VMEM((D_MODEL // 8, 8, IB + 1), jnp.float32),
            pltpu.SemaphoreType.DMA,
            pltpu.SemaphoreType.DMA,
            pltpu.SemaphoreType.DMA,
            pltpu.SemaphoreType.DMA,
            pltpu.SemaphoreType.DMA,
            pltpu.SemaphoreType.DMA,
        ],
        compiler_params=pltpu.CompilerParams(
            use_tc_tiling_on_sc=False, needs_layout_passes=False
        ),
    )
    def k(xp_hbm, table_hbm, out_hbm,
          xv, r0, r1, r2, r3, b0, b1, g0, g1, g2, g3, w0, w1):
        t = lax.axis_index("s") * NC + lax.axis_index("c")
        rows = (r0, r1, r2, r3)
        blks = (b0, b1)
        gsems = (g0, g1, g2, g3)
        wsems = (w0, w1)

        pltpu.sync_copy(xp_hbm.at[:, t], xv)

        iota = lax.iota(jnp.int32, L)

        def fire(j, s):
            pltpu.async_copy(
                table_hbm.at[xv.at[j // nj_lo, j % nj_lo]],
                rows[s],
                gsems[s],
            )

        def drain_gather(s):
            pltpu.make_async_copy(
                table_hbm.at[pl.ds(0, IB)], rows[s], gsems[s]
            ).wait()

        def blk_view(p):
            return blks[p].at[:, :, pl.ds(0, IB)]

        def drain_wb(p):
            pltpu.make_async_copy(
                out_hbm.at[0, :, 0], blk_view(p), wsems[p]
            ).wait()

        def transpose_scale(s, p):
            for kg in range(D_MODEL // L):
                kvec = iota + kg * L
                khi = kvec >> 3
                klo = kvec & 7

                @plsc.parallel_loop(0, IB, unroll=8)
                def _(i):
                    vals = rows[s][i, pl.ds(kg * L, L)]
                    col = jnp.full((L,), i, jnp.int32)
                    plsc.store_scatter(
                        blks[p], [khi, klo, col], vals * SCALE
                    )

        for s in range(RING - 1):
            fire(s, s)

        def outer(g4, carry):
            for r in range(RING):
                j = g4 * RING + r

                @pl.when(j + RING - 1 < nj)
                def _():
                    fire(j + RING - 1, (r + RING - 1) % RING)

                drain_gather(r)
                p = r % 2

                @pl.when(j >= 2)
                def _():
                    drain_wb(p)

                transpose_scale(r, p)
                pltpu.async_copy(blk_view(p), out_hbm.at[j, :, t], wsems[p])
            return carry

        lax.fori_loop(0, nj // RING, outer, 0)
        drain_wb(0)
        drain_wb(1)

    return k(xp, table)


def kernel(x, table):
    n_batch, n_seq = x.shape
    ni = n_batch // IB
    xp = (
        x.T.reshape(n_seq // 8, 8, ni, IB)
        .transpose(0, 2, 1, 3)
        .astype(jnp.int32)
    )
    out5 = _embed(xp, table)
    return out5.transpose(2, 4, 0, 1, 3).reshape(n_batch, n_seq, D_MODEL)
